# Initial kernel scaffold; baseline (speedup 1.0000x reference)
import jax
import jax.numpy as jnp
from jax.experimental import pallas as pl


def kernel(y, edge_x, edge_index, reverse, W_pre, b_pre, W_upd, b_upd):
    raise NotImplementedError("write your pallas kernel here")


# speedup vs baseline: 1.5321x; 1.5321x over previous
import jax
import jax.numpy as jnp
from jax import lax
from jax.experimental import pallas as pl
from jax.experimental.pallas import tpu as pltpu
from jax.experimental.pallas import tpu_sc as plsc

N = 10000
E = 160000
ND = 256
ED = 16
OD = 256
CAT = ND + ED
H = 144
CP = 2 * H
CNTC = CAT - H
NC = 2
NS = 16
L = 16
B = 64
EP = 163840
EPW = EP // NS
NBATCH = EPW // B
CHB = 5
NP = 10240
DUMMY = 10016
RPS = NP // NS
CPS = N // NS
LAST = N - (NS - 1) * RPS


def _pre_node_body(y_ref, w_ref, a0_ref, a1_ref):
    a = jnp.dot(y_ref[...], w_ref[...], preferred_element_type=jnp.float32)
    a0_ref[...] = a[:, :H]
    a1_ref[...] = a[:, H:]


def _pre_edge_body(ex_ref, w_ref, b_ref, b0_ref, b1_ref):
    v = jnp.dot(ex_ref[...], w_ref[...], preferred_element_type=jnp.float32)
    v = v + b_ref[...]
    b0_ref[...] = v[:, :H]
    b1_ref[...] = v[:, H:]


def _sc_body(a0, a1, sdx, bx0, bx1,
             z0, z1,
             acc, sdxb, abuf, bbuf, gsem, ssem, isem):
    c = lax.axis_index("c")
    s = lax.axis_index("s")
    zero = jnp.zeros((L,), jnp.float32)

    def _zrow(i, carry):
        for slot in range(2):
            for j in range(H // L):
                bbuf[slot, i, pl.ds(j * L, L)] = zero
        return carry

    lax.fori_loop(0, B, _zrow, 0)
    for k in range(RPS // B):
        pltpu.sync_copy(bbuf.at[0], acc.at[pl.ds(s * RPS + k * B, B)])
    plsc.subcore_barrier()

    row0 = s * NBATCH

    def _idx_row(b):
        return ((b // CHB) & 1) * CHB + b % CHB

    def _chunk_desc(ch):
        p = (ch & 1) * CHB
        return pltpu.make_async_copy(sdx.at[pl.ds(row0 + ch * CHB, CHB)],
                                     sdxb.at[pl.ds(p, CHB)], isem)

    def _start_gather(b, slot):
        r = _idx_row(b)
        e0 = s * EPW + b * B

        @pl.when(c == 0)
        def _():
            pltpu.make_async_copy(a0.at[sdxb.at[r, 0]], abuf.at[slot],
                                  gsem.at[slot]).start()
            pltpu.make_async_copy(bx0.at[pl.ds(e0, B)], bbuf.at[slot],
                                  gsem.at[slot]).start()

        @pl.when(c == 1)
        def _():
            pltpu.make_async_copy(a1.at[sdxb.at[r, 0]], abuf.at[slot],
                                  gsem.at[slot]).start()
            pltpu.make_async_copy(bx1.at[pl.ds(e0, B)], bbuf.at[slot],
                                  gsem.at[slot]).start()

    def _wait_gather(slot):
        pltpu.make_async_copy(a0.at[pl.ds(0, B)], abuf.at[slot],
                              gsem.at[slot]).wait()
        pltpu.make_async_copy(a0.at[pl.ds(0, B)], bbuf.at[slot],
                              gsem.at[slot]).wait()

    def _scatter_desc(b, slot):
        return pltpu.make_async_copy(bbuf.at[slot], acc.at[sdxb.at[_idx_row(b), 1]],
                                     ssem.at[slot])

    _chunk_desc(0).start()
    _chunk_desc(0).wait()
    _start_gather(0, 0)

    def _pair(g, carry):
        for phase in range(2):
            b = 2 * g + phase
            slot = phase
            n = 1 - phase
            nb = b + 1

            @pl.when(nb < NBATCH)
            def _():
                @pl.when(nb % CHB == 0)
                def _():
                    _chunk_desc(nb // CHB).start()
                    _chunk_desc(nb // CHB).wait()

                @pl.when(b >= 1)
                def _():
                    _scatter_desc(b - 1, n).wait()

                _start_gather(nb, n)

            _wait_gather(slot)

            def _erow(i2, carry2):
                for u in range(2):
                    i = 2 * i2 + u
                    for j in range(H // L):
                        v = abuf[slot, i, pl.ds(j * L, L)] + bbuf[slot, i, pl.ds(j * L, L)]
                        bbuf[slot, i, pl.ds(j * L, L)] = jnp.maximum(v, 0.0)
                return carry2

            lax.fori_loop(0, B // 2, _erow, 0)
            _scatter_desc(b, slot).start(add=True)
        return carry

    lax.fori_loop(0, NBATCH // 2, _pair, 0)
    _scatter_desc(NBATCH - 2, 0).wait()
    _scatter_desc(NBATCH - 1, 1).wait()
    plsc.subcore_barrier()

    r0 = s * CPS

    @pl.when(c == 0)
    def _():
        pltpu.sync_copy(acc.at[pl.ds(r0, CPS)], z0.at[pl.ds(r0, CPS)])

    @pl.when(c == 1)
    def _():
        pltpu.sync_copy(acc.at[pl.ds(r0, CPS)], z1.at[pl.ds(r0, CPS)])


def _update_body(z0_ref, z1_ref, y_ref, w_ref, b_ref, out_ref):
    cnt = jnp.maximum(z1_ref[:, CNTC:CNTC + 1], 1.0)
    z = jnp.concatenate([z0_ref[...], z1_ref[...]], axis=1) / cnt
    h = jnp.dot(z, w_ref[...], preferred_element_type=jnp.float32) + b_ref[...]
    out_ref[...] = jnp.maximum(h, 0.0) + y_ref[...]


def kernel(y, edge_x, edge_index, reverse, W_pre, b_pre, W_upd, b_upd):
    f32 = jnp.float32
    src = jnp.where(reverse, edge_index[1], edge_index[0])
    dst = jnp.where(reverse, edge_index[0], edge_index[1])
    src_p = jnp.concatenate([src, jnp.zeros((EP - E,), jnp.int32)])
    dst_p = jnp.concatenate([dst, jnp.full((EP - E,), DUMMY, jnp.int32)])
    sdx = jnp.stack([src_p.reshape(EP // B, B), dst_p.reshape(EP // B, B)],
                    axis=1)
    ex_p = jnp.concatenate([edge_x, jnp.zeros((EP - E, ED), f32)], axis=0)

    w1 = jnp.pad(W_pre[:ND], ((0, 0), (0, CP - CAT)))
    w2 = jnp.pad(W_pre[ND:], ((0, 0), (0, CP - CAT)))
    bp = jnp.pad(b_pre, (0, CP - CAT)).at[CAT].set(1.0).reshape(1, CP)
    wu = jnp.pad(W_upd, ((0, CP - CAT), (0, 0)))
    bu = b_upd.reshape(1, OD)

    bn = 2000
    a0, a1 = pl.pallas_call(
        _pre_node_body,
        grid=(N // bn,),
        in_specs=[
            pl.BlockSpec((bn, ND), lambda i: (i, 0)),
            pl.BlockSpec((ND, CP), lambda i: (0, 0)),
        ],
        out_specs=[
            pl.BlockSpec((bn, H), lambda i: (i, 0)),
            pl.BlockSpec((bn, H), lambda i: (i, 0)),
        ],
        out_shape=[
            jax.ShapeDtypeStruct((N, H), f32),
            jax.ShapeDtypeStruct((N, H), f32),
        ],
    )(y, w1)

    be = 8192
    bx0, bx1 = pl.pallas_call(
        _pre_edge_body,
        grid=(EP // be,),
        in_specs=[
            pl.BlockSpec((be, ED), lambda i: (i, 0)),
            pl.BlockSpec((ED, CP), lambda i: (0, 0)),
            pl.BlockSpec((1, CP), lambda i: (0, 0)),
        ],
        out_specs=[
            pl.BlockSpec((be, H), lambda i: (i, 0)),
            pl.BlockSpec((be, H), lambda i: (i, 0)),
        ],
        out_shape=[
            jax.ShapeDtypeStruct((EP, H), f32),
            jax.ShapeDtypeStruct((EP, H), f32),
        ],
    )(ex_p, w2, bp)

    sc = pl.kernel(
        _sc_body,
        out_type=(
            jax.ShapeDtypeStruct((N, H), f32),
            jax.ShapeDtypeStruct((N, H), f32),
        ),
        mesh=plsc.VectorSubcoreMesh(core_axis_name="c", subcore_axis_name="s"),
        compiler_params=pltpu.CompilerParams(use_tc_tiling_on_sc=False),
        scratch_types=[
            pltpu.VMEM_SHARED((NP, H), f32),
            pltpu.VMEM((2 * CHB, 2, B), jnp.int32),
            pltpu.VMEM((2, B, H), f32),
            pltpu.VMEM((2, B, H), f32),
            pltpu.SemaphoreType.DMA((2,)),
            pltpu.SemaphoreType.DMA((2,)),
            pltpu.SemaphoreType.DMA,
        ],
    )
    z0, z1 = sc(a0, a1, sdx, bx0, bx1)

    bu_blk = 2000
    h = pl.pallas_call(
        _update_body,
        grid=(N // bu_blk,),
        in_specs=[
            pl.BlockSpec((bu_blk, H), lambda i: (i, 0)),
            pl.BlockSpec((bu_blk, H), lambda i: (i, 0)),
            pl.BlockSpec((bu_blk, ND), lambda i: (i, 0)),
            pl.BlockSpec((CP, OD), lambda i: (0, 0)),
            pl.BlockSpec((1, OD), lambda i: (0, 0)),
        ],
        out_specs=pl.BlockSpec((bu_blk, OD), lambda i: (i, 0)),
        out_shape=jax.ShapeDtypeStruct((N, OD), f32),
    )(z0, z1, y, wu, bu)
    return h


# speedup vs baseline: 1.5331x; 1.0006x over previous
import jax
import jax.numpy as jnp
from jax import lax
from jax.experimental import pallas as pl
from jax.experimental.pallas import tpu as pltpu
from jax.experimental.pallas import tpu_sc as plsc

N = 10000
E = 160000
ND = 256
ED = 16
OD = 256
CAT = ND + ED
H = 144
CP = 2 * H
CNTC = CAT - H
NC = 2
NS = 16
L = 16
B = 64
EP = 163840
EPW = EP // NS
NBATCH = EPW // B
CHB = 5
NP = 10240
DUMMY = 10016
RPS = NP // NS
CPS = N // NS
LAST = N - (NS - 1) * RPS


def _pre_node_body(y_ref, w_ref, a0_ref, a1_ref):
    a = jnp.dot(y_ref[...], w_ref[...], preferred_element_type=jnp.float32)
    a0_ref[...] = a[:, :H]
    a1_ref[...] = a[:, H:]


def _pre_edge_body(ex_ref, w_ref, b_ref, b0_ref, b1_ref):
    v = jnp.dot(ex_ref[...], w_ref[...], preferred_element_type=jnp.float32)
    v = v + b_ref[...]
    b0_ref[...] = v[:, :H]
    b1_ref[...] = v[:, H:]


def _sc_body(a0, a1, sdx, bx0, bx1,
             z0, z1,
             acc, sdxb, abuf, bbuf, gsem, ssem, isem):
    c = lax.axis_index("c")
    s = lax.axis_index("s")
    zero = jnp.zeros((L,), jnp.float32)

    def _zrow(i, carry):
        for slot in range(2):
            for j in range(H // L):
                bbuf[slot, i, pl.ds(j * L, L)] = zero
        return carry

    lax.fori_loop(0, B, _zrow, 0)
    for k in range(RPS // B):
        pltpu.sync_copy(bbuf.at[0], acc.at[pl.ds(s * RPS + k * B, B)])
    plsc.subcore_barrier()

    row0 = s * NBATCH

    def _idx_row(b):
        return ((b // CHB) & 1) * CHB + b % CHB

    def _chunk_desc(ch):
        p = (ch & 1) * CHB
        return pltpu.make_async_copy(sdx.at[pl.ds(row0 + ch * CHB, CHB)],
                                     sdxb.at[pl.ds(p, CHB)], isem)

    def _start_gather(b, slot):
        r = _idx_row(b)
        e0 = s * EPW + b * B

        @pl.when(c == 0)
        def _():
            pltpu.make_async_copy(a0.at[sdxb.at[r, 0]], abuf.at[slot],
                                  gsem.at[slot]).start()
            pltpu.make_async_copy(bx0.at[pl.ds(e0, B)], bbuf.at[slot],
                                  gsem.at[slot]).start()

        @pl.when(c == 1)
        def _():
            pltpu.make_async_copy(a1.at[sdxb.at[r, 0]], abuf.at[slot],
                                  gsem.at[slot]).start()
            pltpu.make_async_copy(bx1.at[pl.ds(e0, B)], bbuf.at[slot],
                                  gsem.at[slot]).start()

    def _wait_gather(slot):
        pltpu.make_async_copy(a0.at[pl.ds(0, B)], abuf.at[slot],
                              gsem.at[slot]).wait()
        pltpu.make_async_copy(a0.at[pl.ds(0, B)], bbuf.at[slot],
                              gsem.at[slot]).wait()

    def _scatter_desc(b, slot):
        return pltpu.make_async_copy(bbuf.at[slot], acc.at[sdxb.at[_idx_row(b), 1]],
                                     ssem.at[slot])

    _chunk_desc(0).start()
    _chunk_desc(0).wait()
    _start_gather(0, 0)

    def _pair(g, carry):
        for phase in range(2):
            b = 2 * g + phase
            slot = phase
            n = 1 - phase
            nb = b + 1

            @pl.when(nb < NBATCH)
            def _():
                @pl.when(nb % CHB == 0)
                def _():
                    _chunk_desc(nb // CHB).wait()

                @pl.when(b >= 1)
                def _():
                    _scatter_desc(b - 1, n).wait()

                _start_gather(nb, n)

            _wait_gather(slot)

            def _erow(i2, carry2):
                for u in range(2):
                    i = 2 * i2 + u
                    for j in range(H // L):
                        v = abuf[slot, i, pl.ds(j * L, L)] + bbuf[slot, i, pl.ds(j * L, L)]
                        bbuf[slot, i, pl.ds(j * L, L)] = jnp.maximum(v, 0.0)
                return carry2

            lax.fori_loop(0, B // 2, _erow, 0)
            _scatter_desc(b, slot).start(add=True)

            @pl.when(((b + 2) % CHB == 0) & (b + 2 < NBATCH))
            def _():
                _chunk_desc((b + 2) // CHB).start()
        return carry

    lax.fori_loop(0, NBATCH // 2, _pair, 0)
    _scatter_desc(NBATCH - 2, 0).wait()
    _scatter_desc(NBATCH - 1, 1).wait()
    plsc.subcore_barrier()

    r0 = s * CPS

    @pl.when(c == 0)
    def _():
        pltpu.sync_copy(acc.at[pl.ds(r0, CPS)], z0.at[pl.ds(r0, CPS)])

    @pl.when(c == 1)
    def _():
        pltpu.sync_copy(acc.at[pl.ds(r0, CPS)], z1.at[pl.ds(r0, CPS)])


def _update_body(z0_ref, z1_ref, y_ref, w_ref, b_ref, out_ref):
    cnt = jnp.maximum(z1_ref[:, CNTC:CNTC + 1], 1.0)
    z = jnp.concatenate([z0_ref[...], z1_ref[...]], axis=1) / cnt
    h = jnp.dot(z, w_ref[...], preferred_element_type=jnp.float32) + b_ref[...]
    out_ref[...] = jnp.maximum(h, 0.0) + y_ref[...]


def kernel(y, edge_x, edge_index, reverse, W_pre, b_pre, W_upd, b_upd):
    f32 = jnp.float32
    src = jnp.where(reverse, edge_index[1], edge_index[0])
    dst = jnp.where(reverse, edge_index[0], edge_index[1])
    src_p = jnp.concatenate([src, jnp.zeros((EP - E,), jnp.int32)])
    dst_p = jnp.concatenate([dst, jnp.full((EP - E,), DUMMY, jnp.int32)])
    sdx = jnp.stack([src_p.reshape(EP // B, B), dst_p.reshape(EP // B, B)],
                    axis=1)
    ex_p = jnp.concatenate([edge_x, jnp.zeros((EP - E, ED), f32)], axis=0)

    w1 = jnp.pad(W_pre[:ND], ((0, 0), (0, CP - CAT)))
    w2 = jnp.pad(W_pre[ND:], ((0, 0), (0, CP - CAT)))
    bp = jnp.pad(b_pre, (0, CP - CAT)).at[CAT].set(1.0).reshape(1, CP)
    wu = jnp.pad(W_upd, ((0, CP - CAT), (0, 0)))
    bu = b_upd.reshape(1, OD)

    bn = 2000
    a0, a1 = pl.pallas_call(
        _pre_node_body,
        grid=(N // bn,),
        in_specs=[
            pl.BlockSpec((bn, ND), lambda i: (i, 0)),
            pl.BlockSpec((ND, CP), lambda i: (0, 0)),
        ],
        out_specs=[
            pl.BlockSpec((bn, H), lambda i: (i, 0)),
            pl.BlockSpec((bn, H), lambda i: (i, 0)),
        ],
        out_shape=[
            jax.ShapeDtypeStruct((N, H), f32),
            jax.ShapeDtypeStruct((N, H), f32),
        ],
    )(y, w1)

    be = 8192
    bx0, bx1 = pl.pallas_call(
        _pre_edge_body,
        grid=(EP // be,),
        in_specs=[
            pl.BlockSpec((be, ED), lambda i: (i, 0)),
            pl.BlockSpec((ED, CP), lambda i: (0, 0)),
            pl.BlockSpec((1, CP), lambda i: (0, 0)),
        ],
        out_specs=[
            pl.BlockSpec((be, H), lambda i: (i, 0)),
            pl.BlockSpec((be, H), lambda i: (i, 0)),
        ],
        out_shape=[
            jax.ShapeDtypeStruct((EP, H), f32),
            jax.ShapeDtypeStruct((EP, H), f32),
        ],
    )(ex_p, w2, bp)

    sc = pl.kernel(
        _sc_body,
        out_type=(
            jax.ShapeDtypeStruct((N, H), f32),
            jax.ShapeDtypeStruct((N, H), f32),
        ),
        mesh=plsc.VectorSubcoreMesh(core_axis_name="c", subcore_axis_name="s"),
        compiler_params=pltpu.CompilerParams(use_tc_tiling_on_sc=False),
        scratch_types=[
            pltpu.VMEM_SHARED((NP, H), f32),
            pltpu.VMEM((2 * CHB, 2, B), jnp.int32),
            pltpu.VMEM((2, B, H), f32),
            pltpu.VMEM((2, B, H), f32),
            pltpu.SemaphoreType.DMA((2,)),
            pltpu.SemaphoreType.DMA((2,)),
            pltpu.SemaphoreType.DMA,
        ],
    )
    z0, z1 = sc(a0, a1, sdx, bx0, bx1)

    bu_blk = 2000
    h = pl.pallas_call(
        _update_body,
        grid=(N // bu_blk,),
        in_specs=[
            pl.BlockSpec((bu_blk, H), lambda i: (i, 0)),
            pl.BlockSpec((bu_blk, H), lambda i: (i, 0)),
            pl.BlockSpec((bu_blk, ND), lambda i: (i, 0)),
            pl.BlockSpec((CP, OD), lambda i: (0, 0)),
            pl.BlockSpec((1, OD), lambda i: (0, 0)),
        ],
        out_specs=pl.BlockSpec((bu_blk, OD), lambda i: (i, 0)),
        out_shape=jax.ShapeDtypeStruct((N, OD), f32),
    )(z0, z1, y, wu, bu)
    return h
